# baseline (speedup 1.0000x reference)
import functools

import jax
import jax.numpy as jnp
from jax import lax
from jax.experimental import pallas as pl
from jax.experimental.pallas import tpu as pltpu
from jax.experimental.pallas import tpu_sc as plsc

D = 64
G = 16384
NC = 2
NS = 16
NW = NC * NS
BPW = G // NW
NCH = 4
CH = BPW // NCH
L = 16


def _build_sc_kernel():
    mesh = plsc.VectorSubcoreMesh(core_axis_name="c", subcore_axis_name="s")

    @functools.partial(
        pl.kernel,
        mesh=mesh,
        out_type=jax.ShapeDtypeStruct((NW, BPW, D), jnp.float32),
        scratch_types=[
            pltpu.VMEM((NCH, CH), jnp.int32),
            pltpu.VMEM((NCH, CH), jnp.int32),
            pltpu.VMEM((BPW,), jnp.int32),
            pltpu.VMEM((BPW,), jnp.float32),
            pltpu.VMEM((2, CH, 128), jnp.float32),
            pltpu.VMEM((2, CH, 128), jnp.float32),
            pltpu.VMEM((CH, D), jnp.float32),
            pltpu.SemaphoreType.DMA,
            pltpu.SemaphoreType.DMA,
            pltpu.SemaphoreType.DMA,
            pltpu.SemaphoreType.DMA,
        ],
    )
    def k(pidx_hbm, midx_hbm, half_hbm, mask_hbm, pe_hbm, mt_hbm, out_hbm,
          pidx_v, midx_v, half_v, mask_v, pe_ring, ms_ring, out_v,
          semp0, semp1, semm0, semm1):
        wid = lax.axis_index("s") * NC + lax.axis_index("c")
        semp = (semp0, semp1)
        semm = (semm0, semm1)

        pltpu.sync_copy(pidx_hbm.at[wid], pidx_v)
        pltpu.sync_copy(midx_hbm.at[wid], midx_v)
        pltpu.sync_copy(half_hbm.at[wid], half_v)
        pltpu.sync_copy(mask_hbm.at[wid], mask_v)

        def fire(c, slot):
            pltpu.async_copy(pe_hbm.at[pidx_v.at[c]], pe_ring.at[slot],
                             semp[slot])
            pltpu.async_copy(mt_hbm.at[midx_v.at[c]], ms_ring.at[slot],
                             semm[slot])

        def wait(slot):
            pltpu.make_async_copy(pe_hbm.at[pidx_v.at[0]], pe_ring.at[slot],
                                  semp[slot]).wait()
            pltpu.make_async_copy(mt_hbm.at[midx_v.at[0]], ms_ring.at[slot],
                                  semm[slot]).wait()

        def combine(c, slot):
            def grp(g, carry):
                base = c * CH + g * L
                svec = half_v[pl.ds(base, L)]
                mvec = mask_v[pl.ds(base, L)]
                for k in range(L):
                    gl = g * L + k
                    s = svec[k]
                    m = mvec[k]
                    for j in range(D // L):
                        pe = pe_ring[slot, gl, pl.ds(s + L * j, L)]
                        ms = ms_ring[slot, gl, pl.ds(L * j, L)]
                        out_v[gl, pl.ds(L * j, L)] = m * (pe - ms) + ms
                return carry

            lax.fori_loop(0, CH // L, grp, 0)
            pltpu.sync_copy(out_v, out_hbm.at[wid, pl.ds(c * CH, CH)])

        fire(0, 0)

        def pair(p, carry):
            ca = 2 * p
            fire(ca + 1, 1)
            wait(0)
            combine(ca, 0)

            @pl.when(p + 1 < NCH // 2)
            def _():
                fire(ca + 2, 0)

            wait(1)
            combine(ca + 1, 1)
            return carry

        lax.fori_loop(0, NCH // 2, pair, 0)

    return k


@jax.jit
def kernel(indices, present_mask, missing_idx_map, pe_table, missing_table):
    idx = indices.astype(jnp.int32)
    pidx = (idx // 2).reshape(NW, NCH, CH)
    half = ((idx % 2) * D).reshape(NW, BPW)
    midx = missing_idx_map.astype(jnp.int32).reshape(NW, NCH, CH)
    mask = present_mask.astype(jnp.float32).reshape(NW, BPW)
    n_missing = missing_table.shape[0]
    mt_ext = jnp.zeros((n_missing, 128), jnp.float32)
    mt_ext = lax.dynamic_update_slice(
        mt_ext, missing_table.astype(jnp.float32), (0, 0))
    pe2 = pe_table.reshape(pe_table.shape[0] // 2, 2 * D)
    out = _build_sc_kernel()(pidx, midx, half, mask, pe2, mt_ext)
    return out.reshape(G, D)


# speedup vs baseline: 2.4767x; 2.4767x over previous
import functools

import jax
import jax.numpy as jnp
from jax import lax
from jax.experimental import pallas as pl
from jax.experimental.pallas import tpu as pltpu
from jax.experimental.pallas import tpu_sc as plsc

D = 64
G = 16384
NC = 2
NS = 16
NW = NC * NS
BPW = G // NW
L = 16
TR = 8
NGRP = BPW // L


def _build_sc_kernel():
    mesh = plsc.VectorSubcoreMesh(core_axis_name="c", subcore_axis_name="s")

    @functools.partial(
        pl.kernel,
        mesh=mesh,
        out_type=jax.ShapeDtypeStruct((NW, BPW, D), jnp.float32),
        scratch_types=[
            pltpu.VMEM((BPW,), jnp.int32),
            pltpu.VMEM((BPW,), jnp.int32),
            pltpu.VMEM((BPW,), jnp.int32),
            pltpu.VMEM((BPW,), jnp.float32),
            pltpu.VMEM((2, L, D), jnp.float32),
            pltpu.VMEM((2, L, 128), jnp.float32),
            pltpu.VMEM((L, D), jnp.float32),
            pltpu.SemaphoreType.DMA,
            pltpu.SemaphoreType.DMA,
            pltpu.SemaphoreType.DMA,
            pltpu.SemaphoreType.DMA,
        ],
    )
    def k(tidx_hbm, sub_hbm, midx_hbm, mask_hbm, pe_hbm, mt_hbm, out_hbm,
          tidx_v, sub_v, midx_v, mask_v, blk_v, ms_v, out_v,
          semp0, semp1, semm0, semm1):
        wid = lax.axis_index("s") * NC + lax.axis_index("c")
        semp = (semp0, semp1)
        semm = (semm0, semm1)

        pltpu.sync_copy(tidx_hbm.at[wid], tidx_v)
        pltpu.sync_copy(sub_hbm.at[wid], sub_v)
        pltpu.sync_copy(midx_hbm.at[wid], midx_v)
        pltpu.sync_copy(mask_hbm.at[wid], mask_v)

        def fire(g, slot):
            tvec = tidx_v[pl.ds(g * L, L)]
            rvec = sub_v[pl.ds(g * L, L)]
            for k in range(L):
                pltpu.async_copy(pe_hbm.at[tvec[k], rvec[k]],
                                 blk_v.at[slot, k], semp[slot])
            pltpu.async_copy(mt_hbm.at[midx_v.at[pl.ds(g * L, L)]],
                             ms_v.at[slot], semm[slot])

        def wait(slot):
            for k in range(L):
                pltpu.make_async_copy(pe_hbm.at[0, 0], blk_v.at[slot, k],
                                      semp[slot]).wait()
            pltpu.make_async_copy(mt_hbm.at[midx_v.at[pl.ds(0, L)]],
                                  ms_v.at[slot], semm[slot]).wait()

        def extract(g, slot):
            mvec = mask_v[pl.ds(g * L, L)]
            for k in range(L):
                m = mvec[k]
                for j in range(D // L):
                    sl = pl.ds(L * j, L)
                    pe = blk_v[slot, k, sl]
                    ms = ms_v[slot, k, sl]
                    out_v[k, sl] = m * (pe - ms) + ms
            pltpu.sync_copy(out_v, out_hbm.at[wid, pl.ds(g * L, L)])

        fire(0, 0)

        def pair(p, carry):
            ga = 2 * p
            fire(ga + 1, 1)
            wait(0)
            extract(ga, 0)

            @pl.when(p + 1 < NGRP // 2)
            def _():
                fire(ga + 2, 0)

            wait(1)
            extract(ga + 1, 1)
            return carry

        lax.fori_loop(0, NGRP // 2, pair, 0)

    return k


@jax.jit
def kernel(indices, present_mask, missing_idx_map, pe_table, missing_table):
    idx = indices.astype(jnp.int32)
    tidx = (idx // TR).reshape(NW, BPW)
    sub = (idx % TR).reshape(NW, BPW)
    midx = missing_idx_map.astype(jnp.int32).reshape(NW, BPW)
    mask = present_mask.astype(jnp.float32).reshape(NW, BPW)
    n_missing = missing_table.shape[0]
    mt_ext = jnp.zeros((n_missing, 128), jnp.float32)
    mt_ext = lax.dynamic_update_slice(
        mt_ext, missing_table.astype(jnp.float32), (0, 0))
    pe3 = pe_table.reshape(pe_table.shape[0] // TR, TR, D)
    out = _build_sc_kernel()(tidx, sub, midx, mask, pe3, mt_ext)
    return out.reshape(G, D)
